# baseline (speedup 1.0000x reference)
import functools

import jax
import jax.numpy as jnp
from jax import lax
from jax.experimental import pallas as pl
from jax.experimental.pallas import tpu as pltpu
from jax.experimental.pallas import tpu_sc as plsc

_NC = 2
_NS = 16
_NW = _NC * _NS

_IDX_PER_DMA = 128
_KB = 4
_CHUNK = _IDX_PER_DMA * _KB


def _sc_gather(x_flat, table):
    n = x_flat.shape[0]
    d = table.shape[1]
    dt = table.dtype
    rows_per_w = n // _NW
    nchunk = rows_per_w // _CHUNK
    assert rows_per_w % _CHUNK == 0 and nchunk % 2 == 0

    x_view = x_flat.reshape(_NW, nchunk, _KB, _IDX_PER_DMA)

    mesh = plsc.VectorSubcoreMesh(core_axis_name="c", subcore_axis_name="s")

    @functools.partial(
        pl.kernel,
        out_type=jax.ShapeDtypeStruct((_NW, nchunk, _CHUNK, d), dt),
        mesh=mesh,
        scratch_types=[
            pltpu.VMEM((2, _KB, _IDX_PER_DMA), jnp.int32),
            pltpu.VMEM((2, _CHUNK, d), dt),
            pltpu.SemaphoreType.DMA,
            pltpu.SemaphoreType.DMA,
        ],
        compiler_params=pltpu.CompilerParams(use_tc_tiling_on_sc=False),
    )
    def gather_kernel(x_hbm, table_hbm, e_hbm, idx_v, rows_v, sem0, sem1):
        wid = lax.axis_index("s") * _NC + lax.axis_index("c")
        sems = (sem0, sem1)

        def body(i, carry):
            handles = []
            for s in range(2):
                chunk = i * 2 + s
                pltpu.sync_copy(x_hbm.at[wid, chunk], idx_v.at[s])
                hs = [
                    pltpu.async_copy(
                        table_hbm.at[idx_v.at[s, j]],
                        rows_v.at[s, pl.ds(j * _IDX_PER_DMA, _IDX_PER_DMA)],
                        sems[s],
                    )
                    for j in range(_KB)
                ]
                handles.append(hs)
            for s in range(2):
                for h in handles[s]:
                    h.wait()
                pltpu.sync_copy(rows_v.at[s], e_hbm.at[wid, i * 2 + s])
            return carry

        lax.fori_loop(0, nchunk // 2, body, 0)

    return gather_kernel(x_view, table).reshape(n, d)


def _tc_project_packed(e3, pe8, w2, b2, t, m):
    ngrp, rows, _ = e3.shape
    gb = 8
    assert ngrp % gb == 0
    bpg = 2 * rows // t
    nbatch = ngrp * bpg

    def mm_kernel(e_ref, pe_ref, w_ref, b_ref, out_ref):
        eb = e_ref[...].astype(jnp.float32) + pe_ref[...][None, :, :]
        acc = jax.lax.dot_general(
            eb.reshape(gb * rows, 128),
            w_ref[...],
            (((1,), (0,)), ((), ())),
            preferred_element_type=jnp.float32,
        )
        out_ref[...] = (acc + b_ref[...]).reshape(gb * bpg, t, m)

    return pl.pallas_call(
        mm_kernel,
        grid=(ngrp // gb,),
        in_specs=[
            pl.BlockSpec((gb, rows, 128), lambda i: (i, 0, 0)),
            pl.BlockSpec((rows, 128), lambda i: (0, 0)),
            pl.BlockSpec((128, 256), lambda i: (0, 0)),
            pl.BlockSpec((1, 256), lambda i: (0, 0)),
        ],
        out_specs=pl.BlockSpec((gb * bpg, t, m), lambda i: (i, 0, 0)),
        out_shape=jax.ShapeDtypeStruct((nbatch, t, m), jnp.float32),
    )(e3, pe8, w2, b2)


def kernel(x, table, pe, W, b):
    if x.ndim == 1:
        x = x[None, :]
    bsz, t = x.shape
    d = table.shape[1]
    m = W.shape[0]

    e_flat = _sc_gather(x.reshape(-1), table.astype(jnp.bfloat16))

    grp = 8
    rows = grp * (t // 2)
    e3 = e_flat.reshape(bsz // grp, rows, 2 * d)
    pe8 = jnp.tile(pe.reshape(t // 2, 2 * d), (grp, 1))

    wt = W.T
    z = jnp.zeros((d, m), jnp.float32)
    w2 = jnp.concatenate(
        [jnp.concatenate([wt, z], 1), jnp.concatenate([z, wt], 1)], 0
    )
    b2 = jnp.concatenate([b, b])[None, :]

    return _tc_project_packed(e3, pe8, w2, b2, t, m)


# speedup vs baseline: 1.5610x; 1.5610x over previous
import functools

import jax
import jax.numpy as jnp
from jax import lax
from jax.experimental import pallas as pl
from jax.experimental.pallas import tpu as pltpu
from jax.experimental.pallas import tpu_sc as plsc

_NC = 2
_NS = 16
_NW = _NC * _NS

_IDX_PER_DMA = 128
_KB = 5
_CHUNK = _IDX_PER_DMA * _KB

_NSPLIT = 4


def _sc_gather(x_flat, table):
    n = x_flat.shape[0]
    d = table.shape[1]
    rows_per_w = n // _NW
    nchunk = rows_per_w // _CHUNK
    assert rows_per_w % _CHUNK == 0 and nchunk % 2 == 0

    x_view = x_flat.reshape(_NW, nchunk, _KB, _IDX_PER_DMA)

    mesh = plsc.VectorSubcoreMesh(core_axis_name="c", subcore_axis_name="s")

    @functools.partial(
        pl.kernel,
        out_type=jax.ShapeDtypeStruct((_NW, nchunk, _CHUNK, d), jnp.float32),
        mesh=mesh,
        scratch_types=[
            pltpu.VMEM((2, _KB, _IDX_PER_DMA), jnp.int32),
            pltpu.VMEM((2, _CHUNK, d), jnp.float32),
            pltpu.SemaphoreType.DMA,
            pltpu.SemaphoreType.DMA,
        ],
        compiler_params=pltpu.CompilerParams(use_tc_tiling_on_sc=False),
    )
    def gather_kernel(x_hbm, table_hbm, e_hbm, idx_v, rows_v, sem0, sem1):
        wid = lax.axis_index("s") * _NC + lax.axis_index("c")
        sems = (sem0, sem1)

        def body(i, carry):
            handles = []
            for s in range(2):
                chunk = i * 2 + s
                pltpu.sync_copy(x_hbm.at[wid, chunk], idx_v.at[s])
                hs = [
                    pltpu.async_copy(
                        table_hbm.at[idx_v.at[s, j]],
                        rows_v.at[s, pl.ds(j * _IDX_PER_DMA, _IDX_PER_DMA)],
                        sems[s],
                    )
                    for j in range(_KB)
                ]
                handles.append(hs)
            for s in range(2):
                for h in handles[s]:
                    h.wait()
                pltpu.sync_copy(rows_v.at[s], e_hbm.at[wid, i * 2 + s])
            return carry

        lax.fori_loop(0, nchunk // 2, body, 0)

    return gather_kernel(x_view, table).reshape(n, d)


def _tc_project_chunk(e3, pe8, w2, b2, t, m, prev, split, nsplit):
    ngrp, rows, _ = e3.shape
    gb = 8
    assert ngrp % gb == 0
    bpg = 2 * rows // t
    nbatch = ngrp * bpg * nsplit
    base = split * (ngrp // gb)

    def mm_kernel(e_ref, pe_ref, w_ref, b_ref, prev_ref, out_ref):
        del prev_ref
        eb = e_ref[...] + pe_ref[...][None, :, :]
        acc = jax.lax.dot_general(
            eb.reshape(gb * rows, 128),
            w_ref[...],
            (((1,), (0,)), ((), ())),
            preferred_element_type=jnp.float32,
        )
        out_ref[...] = (acc + b_ref[...]).reshape(gb * bpg, t, m)

    args = [e3, pe8, w2, b2]
    in_specs = [
        pl.BlockSpec((gb, rows, 128), lambda i: (i, 0, 0)),
        pl.BlockSpec((rows, 128), lambda i: (0, 0)),
        pl.BlockSpec((128, 256), lambda i: (0, 0)),
        pl.BlockSpec((1, 256), lambda i: (0, 0)),
    ]
    kwargs = {}
    if prev is None:
        def mm_kernel0(e_ref, pe_ref, w_ref, b_ref, out_ref):
            mm_kernel(e_ref, pe_ref, w_ref, b_ref, None, out_ref)

        body = mm_kernel0
    else:
        args.append(prev)
        in_specs.append(pl.BlockSpec(memory_space=pl.ANY))
        kwargs["input_output_aliases"] = {4: 0}
        body = mm_kernel

    return pl.pallas_call(
        body,
        grid=(ngrp // gb,),
        in_specs=in_specs,
        out_specs=pl.BlockSpec(
            (gb * bpg, t, m), lambda i: (base + i, 0, 0)
        ),
        out_shape=jax.ShapeDtypeStruct((nbatch, t, m), jnp.float32),
        **kwargs,
    )(*args)


def kernel(x, table, pe, W, b):
    if x.ndim == 1:
        x = x[None, :]
    bsz, t = x.shape
    d = table.shape[1]
    m = W.shape[0]

    tbl = jax.lax.optimization_barrier(table.reshape(-1)).reshape(table.shape)

    grp = 8
    rows = grp * (t // 2)
    pe8 = jnp.tile(pe.reshape(t // 2, 2 * d), (grp, 1))
    wt = W.T
    z = jnp.zeros((d, m), jnp.float32)
    w2 = jnp.concatenate(
        [jnp.concatenate([wt, z], 1), jnp.concatenate([z, wt], 1)], 0
    )
    b2 = jnp.concatenate([b, b])[None, :]

    bchunk = bsz // _NSPLIT
    xf = x.reshape(_NSPLIT, bchunk * t)
    out = None
    for k in range(_NSPLIT):
        e_flat = _sc_gather(xf[k], tbl)
        e3 = e_flat.reshape(bchunk // grp, rows, 2 * d)
        out = _tc_project_chunk(e3, pe8, w2, b2, t, m, out, k, _NSPLIT)
    return out
